# Initial kernel scaffold; baseline (speedup 1.0000x reference)
import jax
import jax.numpy as jnp
from jax.experimental import pallas as pl


def kernel(features, img_size, conv1_w, conv1_b, loc_w, loc_b, score_w, score_b):
    raise NotImplementedError("write your pallas kernel here")


# speedup vs baseline: 164.0578x; 164.0578x over previous
import functools

import numpy as np
import jax
import jax.numpy as jnp
from jax.experimental import pallas as pl

_NMS_THRESH = 0.7
_BSZ = 512


def _generate_anchor_base(base_size=16, ratios=(0.5, 1.0, 2.0), scales=(8, 16, 32)):
    py, px = base_size / 2.0, base_size / 2.0
    ab = np.zeros((len(ratios) * len(scales), 4), dtype=np.float32)
    for i, r in enumerate(ratios):
        for j, s in enumerate(scales):
            h = base_size * s * np.sqrt(r)
            w = base_size * s * np.sqrt(1.0 / r)
            k = i * len(scales) + j
            ab[k, 0] = py - h / 2.0
            ab[k, 1] = px - w / 2.0
            ab[k, 2] = py + h / 2.0
            ab[k, 3] = px + w / 2.0
    return ab


def _make_anchors(h, w, feat_stride=16):
    ab = _generate_anchor_base()
    shift_y = jnp.arange(0, h * feat_stride, feat_stride)
    shift_x = jnp.arange(0, w * feat_stride, feat_stride)
    sx, sy = jnp.meshgrid(shift_x, shift_y, indexing="ij")
    shifts = jnp.stack([sy.ravel(), sx.ravel(), sy.ravel(), sx.ravel()], axis=1).astype(jnp.float32)
    A = ab.shape[0]
    K = shifts.shape[0]
    return (jnp.asarray(ab).reshape(1, A, 4) + shifts.reshape(K, 1, 4)).reshape(K * A, 4)


def _conv2d(x, w, b, padding):
    y = jax.lax.conv_general_dilated(x, w, (1, 1), padding, dimension_numbers=("NCHW", "OIHW", "NCHW"))
    return y + b[None, :, None, None]


def _loc2bbox(anchor, loc):
    ah = anchor[:, 2] - anchor[:, 0]
    aw = anchor[:, 3] - anchor[:, 1]
    acy = anchor[:, 0] + 0.5 * ah
    acx = anchor[:, 1] + 0.5 * aw
    dy, dx, dh, dw = loc[:, 0], loc[:, 1], loc[:, 2], loc[:, 3]
    cy = dy * ah + acy
    cx = dx * aw + acx
    h = jnp.exp(dh) * ah
    w = jnp.exp(dw) * aw
    return jnp.stack([cy - 0.5 * h, cx - 0.5 * w, cy + 0.5 * h, cx + 0.5 * w], axis=1)


def _clip_boxes(roi, img_size):
    y1 = jnp.clip(roi[:, 0], 0, img_size[0])
    x1 = jnp.clip(roi[:, 1], 0, img_size[1])
    y2 = jnp.clip(roi[:, 2], 0, img_size[0])
    x2 = jnp.clip(roi[:, 3], 0, img_size[1])
    return jnp.stack([y1, x1, y2, x2], axis=1)


def _iou_grid(cy1, cx1, cy2, cx2, ca, ry1, rx1, ry2, rx2, ra):
    yy1 = jnp.maximum(cy1, ry1)
    xx1 = jnp.maximum(cx1, rx1)
    yy2 = jnp.minimum(cy2, ry2)
    xx2 = jnp.minimum(cx2, rx2)
    inter = jnp.maximum(yy2 - yy1, 0.0) * jnp.maximum(xx2 - xx1, 0.0)
    return inter / (ca + ra - inter + 1e-9)


def _nms_kernel(nb, bsz, boxes_c_ref, boxes_r_ref, sup_ref):
    sup_ref[...] = jnp.zeros_like(sup_ref)
    ii = jax.lax.broadcasted_iota(jnp.int32, (bsz, bsz), 0)
    jj = jax.lax.broadcasted_iota(jnp.int32, (bsz, bsz), 1)
    tri = (ii < jj).astype(jnp.float32)

    def row_views(t0):
        ry1 = boxes_r_ref[0:1, pl.ds(t0, bsz)]
        rx1 = boxes_r_ref[1:2, pl.ds(t0, bsz)]
        ry2 = boxes_r_ref[2:3, pl.ds(t0, bsz)]
        rx2 = boxes_r_ref[3:4, pl.ds(t0, bsz)]
        ra = (ry2 - ry1) * (rx2 - rx1)
        return ry1, rx1, ry2, rx2, ra

    for b in range(nb):
        s0 = b * bsz
        cy1 = boxes_c_ref[pl.ds(s0, bsz), 0:1]
        cx1 = boxes_c_ref[pl.ds(s0, bsz), 1:2]
        cy2 = boxes_c_ref[pl.ds(s0, bsz), 2:3]
        cx2 = boxes_c_ref[pl.ds(s0, bsz), 3:4]
        ca = (cy2 - cy1) * (cx2 - cx1)
        col = (cy1, cx1, cy2, cx2, ca)

        iou_bb = _iou_grid(*col, *row_views(s0))
        m_bb = jnp.where(iou_bb > _NMS_THRESH, tri, 0.0)
        inc = sup_ref[0:1, pl.ds(s0, bsz)]

        def w_cond(carry):
            return carry[1]

        def w_body(carry):
            s, _ = carry
            act = 1.0 - s
            hits = jax.lax.dot_general(
                act, m_bb, (((1,), (0,)), ((), ())),
                preferred_element_type=jnp.float32)
            s_new = jnp.where(hits > 0.5, 1.0, inc)
            changed = jnp.sum(jnp.abs(s_new - s)) > 0.0
            return (s_new, changed)

        s_fin, _ = jax.lax.while_loop(w_cond, w_body, (inc, True))
        sup_ref[0:1, pl.ds(s0, bsz)] = s_fin
        act_fin = 1.0 - s_fin

        def tail_body(j, _):
            t0 = j * bsz
            iou_bt = _iou_grid(*col, *row_views(t0))
            m_bt = (iou_bt > _NMS_THRESH).astype(jnp.float32)
            hits = jax.lax.dot_general(
                act_fin, m_bt, (((1,), (0,)), ((), ())),
                preferred_element_type=jnp.float32)
            old = sup_ref[0:1, pl.ds(t0, bsz)]
            sup_ref[0:1, pl.ds(t0, bsz)] = jnp.where(hits > 0.5, 1.0, old)
            return 0

        jax.lax.fori_loop(b + 1, nb, tail_body, 0, unroll=False)


def _nms_suppressed_pallas(boxes):
    n = boxes.shape[0]
    npad = ((n + _BSZ - 1) // _BSZ) * _BSZ
    nb = npad // _BSZ
    boxes_p = jnp.zeros((npad, 4), dtype=jnp.float32).at[:n].set(boxes)
    boxes_t = boxes_p.T
    supf = pl.pallas_call(
        functools.partial(_nms_kernel, nb, _BSZ),
        out_shape=jax.ShapeDtypeStruct((1, npad), jnp.float32),
    )(boxes_p, boxes_t)
    return supf[0, :n] > 0.5


def _proposal_creator(loc, score, anchors, img_size, n_pre=12000, n_post=2000,
                      nms_thresh=0.7, min_size=16.0):
    roi = _clip_boxes(_loc2bbox(anchors, loc), img_size)
    hs = roi[:, 2] - roi[:, 0]
    ws = roi[:, 3] - roi[:, 1]
    valid = (hs >= min_size) & (ws >= min_size)
    s = jnp.where(valid, score, -1e9)
    n_pre = min(n_pre, roi.shape[0])
    vals, order = jax.lax.top_k(s, n_pre)
    boxes_s = roi[order]
    sup = _nms_suppressed_pallas(boxes_s)
    keep_scores = jnp.where(sup, -1e9, vals)
    n_post = min(n_post, n_pre)
    _, keep_idx = jax.lax.top_k(keep_scores, n_post)
    return boxes_s[keep_idx], order, keep_idx


def kernel(features, img_size, conv1_w, conv1_b, loc_w, loc_b, score_w, score_b):
    n, _, h, w = features.shape
    anchors = _make_anchors(h, w)
    x = jax.nn.relu(_conv2d(features, conv1_w, conv1_b, "SAME"))
    rpn_locs = jnp.transpose(_conv2d(x, loc_w, loc_b, "VALID"), (0, 2, 3, 1)).reshape(n, -1, 4)
    rpn_scores = jnp.transpose(_conv2d(x, score_w, score_b, "VALID"), (0, 2, 3, 1)).reshape(n, -1, 2)
    fg = jax.nn.softmax(rpn_scores, axis=2)[:, :, 1]
    rois = []
    roi_inds = []
    for i in range(n):
        roi, _, _ = _proposal_creator(rpn_locs[i], fg[i], anchors, img_size)
        rois.append(roi)
        roi_inds.append(jnp.full((roi.shape[0],), i, dtype=jnp.float32))
    return rpn_locs, rpn_scores, jnp.concatenate(rois, axis=0), jnp.concatenate(roi_inds, axis=0), anchors


# speedup vs baseline: 196.6336x; 1.1986x over previous
import functools

import numpy as np
import jax
import jax.numpy as jnp
from jax.experimental import pallas as pl
from jax.experimental.pallas import tpu as pltpu

_NMS_THRESH = 0.7
_BSZ = 512


def _generate_anchor_base(base_size=16, ratios=(0.5, 1.0, 2.0), scales=(8, 16, 32)):
    py, px = base_size / 2.0, base_size / 2.0
    ab = np.zeros((len(ratios) * len(scales), 4), dtype=np.float32)
    for i, r in enumerate(ratios):
        for j, s in enumerate(scales):
            h = base_size * s * np.sqrt(r)
            w = base_size * s * np.sqrt(1.0 / r)
            k = i * len(scales) + j
            ab[k, 0] = py - h / 2.0
            ab[k, 1] = px - w / 2.0
            ab[k, 2] = py + h / 2.0
            ab[k, 3] = px + w / 2.0
    return ab


def _make_anchors(h, w, feat_stride=16):
    ab = _generate_anchor_base()
    shift_y = jnp.arange(0, h * feat_stride, feat_stride)
    shift_x = jnp.arange(0, w * feat_stride, feat_stride)
    sx, sy = jnp.meshgrid(shift_x, shift_y, indexing="ij")
    shifts = jnp.stack([sy.ravel(), sx.ravel(), sy.ravel(), sx.ravel()], axis=1).astype(jnp.float32)
    A = ab.shape[0]
    K = shifts.shape[0]
    return (jnp.asarray(ab).reshape(1, A, 4) + shifts.reshape(K, 1, 4)).reshape(K * A, 4)


def _conv2d(x, w, b, padding):
    y = jax.lax.conv_general_dilated(x, w, (1, 1), padding, dimension_numbers=("NCHW", "OIHW", "NCHW"))
    return y + b[None, :, None, None]


def _loc2bbox(anchor, loc):
    ah = anchor[:, 2] - anchor[:, 0]
    aw = anchor[:, 3] - anchor[:, 1]
    acy = anchor[:, 0] + 0.5 * ah
    acx = anchor[:, 1] + 0.5 * aw
    dy, dx, dh, dw = loc[:, 0], loc[:, 1], loc[:, 2], loc[:, 3]
    cy = dy * ah + acy
    cx = dx * aw + acx
    h = jnp.exp(dh) * ah
    w = jnp.exp(dw) * aw
    return jnp.stack([cy - 0.5 * h, cx - 0.5 * w, cy + 0.5 * h, cx + 0.5 * w], axis=1)


def _clip_boxes(roi, img_size):
    y1 = jnp.clip(roi[:, 0], 0, img_size[0])
    x1 = jnp.clip(roi[:, 1], 0, img_size[1])
    y2 = jnp.clip(roi[:, 2], 0, img_size[0])
    x2 = jnp.clip(roi[:, 3], 0, img_size[1])
    return jnp.stack([y1, x1, y2, x2], axis=1)


def _decode_kernel(locs_ref, sco_ref, anch_ref, img_ref, roi_ref, s_ref):
    ay1 = anch_ref[0:1, :]
    ax1 = anch_ref[1:2, :]
    ay2 = anch_ref[2:3, :]
    ax2 = anch_ref[3:4, :]
    ah = ay2 - ay1
    aw = ax2 - ax1
    acy = ay1 + 0.5 * ah
    acx = ax1 + 0.5 * aw
    dy = locs_ref[0:1, :]
    dx = locs_ref[1:2, :]
    dh = locs_ref[2:3, :]
    dw = locs_ref[3:4, :]
    cy = dy * ah + acy
    cx = dx * aw + acx
    h = jnp.exp(dh) * ah
    w = jnp.exp(dw) * aw
    img_h = img_ref[0]
    img_w = img_ref[1]
    y1 = jnp.clip(cy - 0.5 * h, 0, img_h)
    x1 = jnp.clip(cx - 0.5 * w, 0, img_w)
    y2 = jnp.clip(cy + 0.5 * h, 0, img_h)
    x2 = jnp.clip(cx + 0.5 * w, 0, img_w)
    roi_ref[0:1, :] = y1
    roi_ref[1:2, :] = x1
    roi_ref[2:3, :] = y2
    roi_ref[3:4, :] = x2
    s0 = sco_ref[0:1, :]
    s1 = sco_ref[1:2, :]
    m = jnp.maximum(s0, s1)
    e0 = jnp.exp(s0 - m)
    e1 = jnp.exp(s1 - m)
    fg = e1 / (e0 + e1)
    valid = ((y2 - y1) >= 16.0) & ((x2 - x1) >= 16.0)
    s_ref[...] = jnp.where(valid, fg, -1e9)


def _decode_pallas(locsT, scoresT, anchorsT, img_size):
    n = locsT.shape[1]
    roiT, s = pl.pallas_call(
        _decode_kernel,
        out_shape=(
            jax.ShapeDtypeStruct((4, n), jnp.float32),
            jax.ShapeDtypeStruct((1, n), jnp.float32),
        ),
        in_specs=[
            pl.BlockSpec(memory_space=pltpu.VMEM),
            pl.BlockSpec(memory_space=pltpu.VMEM),
            pl.BlockSpec(memory_space=pltpu.VMEM),
            pl.BlockSpec(memory_space=pltpu.SMEM),
        ],
        out_specs=(
            pl.BlockSpec(memory_space=pltpu.VMEM),
            pl.BlockSpec(memory_space=pltpu.VMEM),
        ),
    )(locsT, scoresT, anchorsT, img_size)
    return roiT, s


_SCOLS = 128


def _sort_kernel(rows, keys_ref, keys_out, idx_out):
    ntot = rows * _SCOLS
    r_iota = jax.lax.broadcasted_iota(jnp.int32, (rows, _SCOLS), 0)
    c_iota = jax.lax.broadcasted_iota(jnp.int32, (rows, _SCOLS), 1)
    keys = keys_ref[...]
    idxs = r_iota * _SCOLS + c_iota

    def substage(kk, jj, keys, idxs):
        if jj >= _SCOLS:
            d, axis, size = jj // _SCOLS, 0, rows
            lower = (r_iota & d) == 0
        else:
            d, axis, size = jj, 1, _SCOLS
            lower = (c_iota & d) == 0
        if kk >= _SCOLS:
            up = (r_iota & (kk // _SCOLS)) == 0
        else:
            up = (c_iota & kk) == 0
        pk = jnp.where(lower, pltpu.roll(keys, size - d, axis), pltpu.roll(keys, d, axis))
        pi = jnp.where(lower, pltpu.roll(idxs, size - d, axis), pltpu.roll(idxs, d, axis))
        win = (keys > pk) | ((keys == pk) & (idxs < pi))
        choose_self = win == (lower == up)
        return jnp.where(choose_self, keys, pk), jnp.where(choose_self, idxs, pi)

    kk = 2
    while kk <= ntot:
        jj = kk // 2
        while jj >= 1:
            keys, idxs = substage(kk, jj, keys, idxs)
            jj //= 2
        kk *= 2
    keys_out[...] = keys
    idx_out[...] = idxs


def _argsort_desc_pallas(scores_flat):
    n = scores_flat.shape[0]
    ntot = _SCOLS
    while ntot < n:
        ntot *= 2
    rows = ntot // _SCOLS
    pad = jnp.full((ntot - n,), -jnp.inf, dtype=jnp.float32)
    keys = jnp.concatenate([scores_flat, pad]).reshape(rows, _SCOLS)
    ks, ix = pl.pallas_call(
        functools.partial(_sort_kernel, rows),
        out_shape=(
            jax.ShapeDtypeStruct((rows, _SCOLS), jnp.float32),
            jax.ShapeDtypeStruct((rows, _SCOLS), jnp.int32),
        ),
    )(keys)
    return ks.reshape(-1)[:n], ix.reshape(-1)[:n]


def _iou_grid(cy1, cx1, cy2, cx2, ca, ry1, rx1, ry2, rx2, ra):
    yy1 = jnp.maximum(cy1, ry1)
    xx1 = jnp.maximum(cx1, rx1)
    yy2 = jnp.minimum(cy2, ry2)
    xx2 = jnp.minimum(cx2, rx2)
    inter = jnp.maximum(yy2 - yy1, 0.0) * jnp.maximum(xx2 - xx1, 0.0)
    return inter / (ca + ra - inter + 1e-9)


def _nms_kernel(nb, bsz, boxes_c_ref, boxes_r_ref, sup_ref):
    sup_ref[...] = jnp.zeros_like(sup_ref)
    ii = jax.lax.broadcasted_iota(jnp.int32, (bsz, bsz), 0)
    jj = jax.lax.broadcasted_iota(jnp.int32, (bsz, bsz), 1)
    tri = (ii < jj).astype(jnp.float32)

    def row_views(t0):
        ry1 = boxes_r_ref[0:1, pl.ds(t0, bsz)]
        rx1 = boxes_r_ref[1:2, pl.ds(t0, bsz)]
        ry2 = boxes_r_ref[2:3, pl.ds(t0, bsz)]
        rx2 = boxes_r_ref[3:4, pl.ds(t0, bsz)]
        ra = (ry2 - ry1) * (rx2 - rx1)
        return ry1, rx1, ry2, rx2, ra

    for b in range(nb):
        s0 = b * bsz
        cy1 = boxes_c_ref[pl.ds(s0, bsz), 0:1]
        cx1 = boxes_c_ref[pl.ds(s0, bsz), 1:2]
        cy2 = boxes_c_ref[pl.ds(s0, bsz), 2:3]
        cx2 = boxes_c_ref[pl.ds(s0, bsz), 3:4]
        ca = (cy2 - cy1) * (cx2 - cx1)
        col = (cy1, cx1, cy2, cx2, ca)

        iou_bb = _iou_grid(*col, *row_views(s0))
        m_bb = jnp.where(iou_bb > _NMS_THRESH, tri, 0.0)
        inc = sup_ref[0:1, pl.ds(s0, bsz)]

        def w_cond(carry):
            return carry[1]

        def w_body(carry):
            s, _ = carry
            act = 1.0 - s
            hits = jax.lax.dot_general(
                act, m_bb, (((1,), (0,)), ((), ())),
                preferred_element_type=jnp.float32)
            s_new = jnp.where(hits > 0.5, 1.0, inc)
            changed = jnp.sum(jnp.abs(s_new - s)) > 0.0
            return (s_new, changed)

        s_fin, _ = jax.lax.while_loop(w_cond, w_body, (inc, True))
        sup_ref[0:1, pl.ds(s0, bsz)] = s_fin
        act_fin = 1.0 - s_fin

        def tail_body(j, _):
            t0 = j * bsz
            iou_bt = _iou_grid(*col, *row_views(t0))
            m_bt = (iou_bt > _NMS_THRESH).astype(jnp.float32)
            hits = jax.lax.dot_general(
                act_fin, m_bt, (((1,), (0,)), ((), ())),
                preferred_element_type=jnp.float32)
            old = sup_ref[0:1, pl.ds(t0, bsz)]
            sup_ref[0:1, pl.ds(t0, bsz)] = jnp.where(hits > 0.5, 1.0, old)
            return 0

        jax.lax.fori_loop(b + 1, nb, tail_body, 0, unroll=False)


def _nms_suppressed_pallas(boxes):
    n = boxes.shape[0]
    npad = ((n + _BSZ - 1) // _BSZ) * _BSZ
    nb = npad // _BSZ
    boxes_p = jnp.zeros((npad, 4), dtype=jnp.float32).at[:n].set(boxes)
    boxes_t = boxes_p.T
    supf = pl.pallas_call(
        functools.partial(_nms_kernel, nb, _BSZ),
        out_shape=jax.ShapeDtypeStruct((1, npad), jnp.float32),
    )(boxes_p, boxes_t)
    return supf[0, :n] > 0.5


def _proposal_creator(locsT, scoresT, anchorsT, img_size, n_pre=12000, n_post=2000,
                      nms_thresh=0.7, min_size=16.0):
    roiT, s2 = _decode_pallas(locsT, scoresT, anchorsT, img_size)
    roi = roiT.T
    s = s2[0]
    n_pre = min(n_pre, roi.shape[0])
    vals_all, order_all = _argsort_desc_pallas(s)
    vals, order = vals_all[:n_pre], order_all[:n_pre]
    del vals_all, order_all
    boxes_s = roi[order]
    sup = _nms_suppressed_pallas(boxes_s)
    keep_scores = jnp.where(sup, -1e9, vals)
    n_post = min(n_post, n_pre)
    _, keep_all = _argsort_desc_pallas(keep_scores)
    keep_idx = keep_all[:n_post]
    return boxes_s[keep_idx], order, keep_idx


def kernel(features, img_size, conv1_w, conv1_b, loc_w, loc_b, score_w, score_b):
    n, _, h, w = features.shape
    anchors = _make_anchors(h, w)
    x = jax.nn.relu(_conv2d(features, conv1_w, conv1_b, "SAME"))
    rpn_locs = jnp.transpose(_conv2d(x, loc_w, loc_b, "VALID"), (0, 2, 3, 1)).reshape(n, -1, 4)
    rpn_scores = jnp.transpose(_conv2d(x, score_w, score_b, "VALID"), (0, 2, 3, 1)).reshape(n, -1, 2)
    anchorsT = anchors.T
    rois = []
    roi_inds = []
    for i in range(n):
        roi, _, _ = _proposal_creator(rpn_locs[i].T, rpn_scores[i].T, anchorsT, img_size)
        rois.append(roi)
        roi_inds.append(jnp.full((roi.shape[0],), i, dtype=jnp.float32))
    return rpn_locs, rpn_scores, jnp.concatenate(rois, axis=0), jnp.concatenate(roi_inds, axis=0), anchors


# speedup vs baseline: 198.9704x; 1.0119x over previous
import functools

import numpy as np
import jax
import jax.numpy as jnp
from jax.experimental import pallas as pl
from jax.experimental.pallas import tpu as pltpu
from jax.experimental.pallas import tpu_sc as plsc

_NMS_THRESH = 0.7
_BSZ = 512


def _generate_anchor_base(base_size=16, ratios=(0.5, 1.0, 2.0), scales=(8, 16, 32)):
    py, px = base_size / 2.0, base_size / 2.0
    ab = np.zeros((len(ratios) * len(scales), 4), dtype=np.float32)
    for i, r in enumerate(ratios):
        for j, s in enumerate(scales):
            h = base_size * s * np.sqrt(r)
            w = base_size * s * np.sqrt(1.0 / r)
            k = i * len(scales) + j
            ab[k, 0] = py - h / 2.0
            ab[k, 1] = px - w / 2.0
            ab[k, 2] = py + h / 2.0
            ab[k, 3] = px + w / 2.0
    return ab


def _make_anchors(h, w, feat_stride=16):
    ab = _generate_anchor_base()
    shift_y = jnp.arange(0, h * feat_stride, feat_stride)
    shift_x = jnp.arange(0, w * feat_stride, feat_stride)
    sx, sy = jnp.meshgrid(shift_x, shift_y, indexing="ij")
    shifts = jnp.stack([sy.ravel(), sx.ravel(), sy.ravel(), sx.ravel()], axis=1).astype(jnp.float32)
    A = ab.shape[0]
    K = shifts.shape[0]
    return (jnp.asarray(ab).reshape(1, A, 4) + shifts.reshape(K, 1, 4)).reshape(K * A, 4)


def _conv2d(x, w, b, padding):
    y = jax.lax.conv_general_dilated(x, w, (1, 1), padding, dimension_numbers=("NCHW", "OIHW", "NCHW"))
    return y + b[None, :, None, None]


def _decode_kernel(locs_ref, sco_ref, anch_ref, img_ref, roi_ref, s_ref):
    ay1 = anch_ref[0:1, :]
    ax1 = anch_ref[1:2, :]
    ay2 = anch_ref[2:3, :]
    ax2 = anch_ref[3:4, :]
    ah = ay2 - ay1
    aw = ax2 - ax1
    acy = ay1 + 0.5 * ah
    acx = ax1 + 0.5 * aw
    dy = locs_ref[0:1, :]
    dx = locs_ref[1:2, :]
    dh = locs_ref[2:3, :]
    dw = locs_ref[3:4, :]
    cy = dy * ah + acy
    cx = dx * aw + acx
    h = jnp.exp(dh) * ah
    w = jnp.exp(dw) * aw
    img_h = img_ref[0]
    img_w = img_ref[1]
    y1 = jnp.clip(cy - 0.5 * h, 0, img_h)
    x1 = jnp.clip(cx - 0.5 * w, 0, img_w)
    y2 = jnp.clip(cy + 0.5 * h, 0, img_h)
    x2 = jnp.clip(cx + 0.5 * w, 0, img_w)
    roi_ref[0:1, :] = y1
    roi_ref[1:2, :] = x1
    roi_ref[2:3, :] = y2
    roi_ref[3:4, :] = x2
    s0 = sco_ref[0:1, :]
    s1 = sco_ref[1:2, :]
    m = jnp.maximum(s0, s1)
    e0 = jnp.exp(s0 - m)
    e1 = jnp.exp(s1 - m)
    fg = e1 / (e0 + e1)
    valid = ((y2 - y1) >= 16.0) & ((x2 - x1) >= 16.0)
    s_ref[...] = jnp.where(valid, fg, -1e9)


def _decode_pallas(locsT, scoresT, anchorsT, img_size):
    n = locsT.shape[1]
    roiT, s = pl.pallas_call(
        _decode_kernel,
        out_shape=(
            jax.ShapeDtypeStruct((4, n), jnp.float32),
            jax.ShapeDtypeStruct((1, n), jnp.float32),
        ),
        in_specs=[
            pl.BlockSpec(memory_space=pltpu.VMEM),
            pl.BlockSpec(memory_space=pltpu.VMEM),
            pl.BlockSpec(memory_space=pltpu.VMEM),
            pl.BlockSpec(memory_space=pltpu.SMEM),
        ],
        out_specs=(
            pl.BlockSpec(memory_space=pltpu.VMEM),
            pl.BlockSpec(memory_space=pltpu.VMEM),
        ),
    )(locsT, scoresT, anchorsT, img_size)
    return roiT, s


_SCOLS = 128


def _sort_kernel(rows, keys_ref, keys_out, idx_out):
    ntot = rows * _SCOLS
    r_iota = jax.lax.broadcasted_iota(jnp.int32, (rows, _SCOLS), 0)
    c_iota = jax.lax.broadcasted_iota(jnp.int32, (rows, _SCOLS), 1)
    keys = keys_ref[...]
    idxs = r_iota * _SCOLS + c_iota

    def substage(kk, jj, keys, idxs):
        if jj >= _SCOLS:
            d, axis, size = jj // _SCOLS, 0, rows
            lower = (r_iota & d) == 0
        else:
            d, axis, size = jj, 1, _SCOLS
            lower = (c_iota & d) == 0
        if kk >= _SCOLS:
            up = (r_iota & (kk // _SCOLS)) == 0
        else:
            up = (c_iota & kk) == 0
        pk = jnp.where(lower, pltpu.roll(keys, size - d, axis), pltpu.roll(keys, d, axis))
        pi = jnp.where(lower, pltpu.roll(idxs, size - d, axis), pltpu.roll(idxs, d, axis))
        win = (keys > pk) | ((keys == pk) & (idxs < pi))
        choose_self = win == (lower == up)
        return jnp.where(choose_self, keys, pk), jnp.where(choose_self, idxs, pi)

    kk = 2
    while kk <= ntot:
        jj = kk // 2
        while jj >= 1:
            keys, idxs = substage(kk, jj, keys, idxs)
            jj //= 2
        kk *= 2
    keys_out[...] = keys
    idx_out[...] = idxs


def _argsort_desc_pallas(scores_flat):
    n = scores_flat.shape[0]
    ntot = _SCOLS
    while ntot < n:
        ntot *= 2
    rows = ntot // _SCOLS
    pad = jnp.full((ntot - n,), -jnp.inf, dtype=jnp.float32)
    keys = jnp.concatenate([scores_flat, pad]).reshape(rows, _SCOLS)
    ks, ix = pl.pallas_call(
        functools.partial(_sort_kernel, rows),
        out_shape=(
            jax.ShapeDtypeStruct((rows, _SCOLS), jnp.float32),
            jax.ShapeDtypeStruct((rows, _SCOLS), jnp.int32),
        ),
    )(keys)
    return ks.reshape(-1)[:n], ix.reshape(-1)[:n]


def _iou_grid(cy1, cx1, cy2, cx2, ca, ry1, rx1, ry2, rx2, ra):
    yy1 = jnp.maximum(cy1, ry1)
    xx1 = jnp.maximum(cx1, rx1)
    yy2 = jnp.minimum(cy2, ry2)
    xx2 = jnp.minimum(cx2, rx2)
    inter = jnp.maximum(yy2 - yy1, 0.0) * jnp.maximum(xx2 - xx1, 0.0)
    return inter / (ca + ra - inter + 1e-9)


def _nms_kernel(nb, bsz, boxes_c_ref, boxes_r_ref, sup_ref):
    sup_ref[...] = jnp.zeros_like(sup_ref)
    ii = jax.lax.broadcasted_iota(jnp.int32, (bsz, bsz), 0)
    jj = jax.lax.broadcasted_iota(jnp.int32, (bsz, bsz), 1)
    tri = (ii < jj).astype(jnp.float32)

    def row_views(t0):
        ry1 = boxes_r_ref[0:1, pl.ds(t0, bsz)]
        rx1 = boxes_r_ref[1:2, pl.ds(t0, bsz)]
        ry2 = boxes_r_ref[2:3, pl.ds(t0, bsz)]
        rx2 = boxes_r_ref[3:4, pl.ds(t0, bsz)]
        ra = (ry2 - ry1) * (rx2 - rx1)
        return ry1, rx1, ry2, rx2, ra

    for b in range(nb):
        s0 = b * bsz
        cy1 = boxes_c_ref[pl.ds(s0, bsz), 0:1]
        cx1 = boxes_c_ref[pl.ds(s0, bsz), 1:2]
        cy2 = boxes_c_ref[pl.ds(s0, bsz), 2:3]
        cx2 = boxes_c_ref[pl.ds(s0, bsz), 3:4]
        ca = (cy2 - cy1) * (cx2 - cx1)
        col = (cy1, cx1, cy2, cx2, ca)

        iou_bb = _iou_grid(*col, *row_views(s0))
        m_bb = jnp.where(iou_bb > _NMS_THRESH, tri, 0.0)
        inc = sup_ref[0:1, pl.ds(s0, bsz)]

        def w_cond(carry):
            return carry[1]

        def w_body(carry):
            s, _ = carry
            act = 1.0 - s
            hits = jax.lax.dot_general(
                act, m_bb, (((1,), (0,)), ((), ())),
                preferred_element_type=jnp.float32)
            s_new = jnp.where(hits > 0.5, 1.0, inc)
            changed = jnp.sum(jnp.abs(s_new - s)) > 0.0
            return (s_new, changed)

        s_fin, _ = jax.lax.while_loop(w_cond, w_body, (inc, True))
        sup_ref[0:1, pl.ds(s0, bsz)] = s_fin
        act_fin = 1.0 - s_fin

        def tail_body(j, _):
            t0 = j * bsz
            iou_bt = _iou_grid(*col, *row_views(t0))
            m_bt = (iou_bt > _NMS_THRESH).astype(jnp.float32)
            hits = jax.lax.dot_general(
                act_fin, m_bt, (((1,), (0,)), ((), ())),
                preferred_element_type=jnp.float32)
            old = sup_ref[0:1, pl.ds(t0, bsz)]
            sup_ref[0:1, pl.ds(t0, bsz)] = jnp.where(hits > 0.5, 1.0, old)
            return 0

        jax.lax.fori_loop(b + 1, nb, tail_body, 0, unroll=False)


def _sc_row_gather_t(table_t, idx):
    b = idx.shape[0]
    n = table_t.shape[1]
    try:
        info = plsc.get_sparse_core_info()
        nc, ns = info.num_cores, info.num_subcores
    except Exception:
        nc, ns = 2, 16
    nw = nc * ns
    bpw = b // nw
    mesh = plsc.VectorSubcoreMesh(core_axis_name="c", subcore_axis_name="s")

    @functools.partial(
        pl.kernel,
        mesh=mesh,
        out_type=jax.ShapeDtypeStruct((4, b), jnp.float32),
        compiler_params=pltpu.CompilerParams(needs_layout_passes=False),
        scratch_types=[
            pltpu.VMEM((n,), jnp.float32),
            pltpu.VMEM((bpw,), jnp.int32),
            pltpu.VMEM((bpw,), jnp.float32),
        ],
    )
    def k(tbl_hbm, idx_hbm, out_hbm, table_v, idx_v, outc_v):
        wid = jax.lax.axis_index("s") * nc + jax.lax.axis_index("c")
        base = wid * bpw
        pltpu.sync_copy(idx_hbm.at[pl.ds(base, bpw)], idx_v)
        for c in range(4):
            pltpu.sync_copy(tbl_hbm.at[c], table_v)
            for j in range(bpw // 16):
                ii = idx_v[pl.ds(j * 16, 16)]
                outc_v[pl.ds(j * 16, 16)] = plsc.load_gather(table_v, [ii])
            pltpu.sync_copy(outc_v, out_hbm.at[c, pl.ds(base, bpw)])

    return k(table_t, idx)


def _nms_suppressed_pallas(boxes_p, boxes_t, n):
    npad = boxes_p.shape[0]
    nb = npad // _BSZ
    supf = pl.pallas_call(
        functools.partial(_nms_kernel, nb, _BSZ),
        out_shape=jax.ShapeDtypeStruct((1, npad), jnp.float32),
    )(boxes_p, boxes_t)
    return supf[0, :n] > 0.5


def _proposal_creator(locsT, scoresT, anchorsT, img_size, n_pre=12000, n_post=2000,
                      nms_thresh=0.7, min_size=16.0):
    roiT, s2 = _decode_pallas(locsT, scoresT, anchorsT, img_size)
    s = s2[0]
    n_pre = min(n_pre, s.shape[0])
    vals_all, order_all = _argsort_desc_pallas(s)
    vals, order = vals_all[:n_pre], order_all[:n_pre]
    npad = ((n_pre + _BSZ - 1) // _BSZ) * _BSZ
    order_pad = jnp.concatenate([order, jnp.zeros((npad - n_pre,), jnp.int32)])
    boxes_t = _sc_row_gather_t(roiT, order_pad)
    boxes_pad = boxes_t.T
    sup = _nms_suppressed_pallas(boxes_pad, boxes_t, n_pre)
    keep_scores = jnp.where(sup, -1e9, vals)
    n_post = min(n_post, n_pre)
    _, keep_all = _argsort_desc_pallas(keep_scores)
    keep_idx = keep_all[:n_post]
    kpad = 2048
    keep_pad = jnp.concatenate([keep_idx, jnp.zeros((kpad - n_post,), jnp.int32)])
    rois_out = _sc_row_gather_t(boxes_t, keep_pad).T[:n_post]
    return rois_out, order, keep_idx


def kernel(features, img_size, conv1_w, conv1_b, loc_w, loc_b, score_w, score_b):
    n, _, h, w = features.shape
    anchors = _make_anchors(h, w)
    x = jax.nn.relu(_conv2d(features, conv1_w, conv1_b, "SAME"))
    rpn_locs = jnp.transpose(_conv2d(x, loc_w, loc_b, "VALID"), (0, 2, 3, 1)).reshape(n, -1, 4)
    rpn_scores = jnp.transpose(_conv2d(x, score_w, score_b, "VALID"), (0, 2, 3, 1)).reshape(n, -1, 2)
    anchorsT = anchors.T
    rois = []
    roi_inds = []
    for i in range(n):
        roi, _, _ = _proposal_creator(rpn_locs[i].T, rpn_scores[i].T, anchorsT, img_size)
        rois.append(roi)
        roi_inds.append(jnp.full((roi.shape[0],), i, dtype=jnp.float32))
    return rpn_locs, rpn_scores, jnp.concatenate(rois, axis=0), jnp.concatenate(roi_inds, axis=0), anchors
